# baseline (speedup 1.0000x reference)
import functools

import jax
import jax.numpy as jnp
from jax import lax
from jax.experimental import pallas as pl
from jax.experimental.pallas import tpu as pltpu
from jax.experimental.pallas import tpu_sc as plsc

_VOCAB = 1000000
_D = 64
_BATCH = 4096
_HIST = 200

_NC, _NS = 2, 16
_NW = _NC * _NS
_B = _BATCH * _HIST
_BPW = _B // _NW
_CH = 128
_NCH = _BPW // _CH
_G = 8
_NR = _NCH // _G

_mesh = plsc.VectorSubcoreMesh(core_axis_name="c", subcore_axis_name="s")


@functools.partial(
    pl.kernel,
    mesh=_mesh,
    out_type=jax.ShapeDtypeStruct((_B, _D), jnp.float32),
    scratch_types=[
        pltpu.VMEM((_NCH, _CH), jnp.int32),
        pltpu.VMEM((_G, _CH, _D), jnp.float32),
        pltpu.SemaphoreType.DMA,
        pltpu.SemaphoreType.DMA,
    ],
    compiler_params=pltpu.CompilerParams(use_tc_tiling_on_sc=False),
)
def _sc_gather(idx_hbm, table_hbm, out_hbm, idx_v, rows_v, gsem, ssem):
    wid = lax.axis_index("s") * _NC + lax.axis_index("c")
    base = wid * _BPW
    pltpu.sync_copy(idx_hbm.at[wid], idx_v)

    def round_body(r, carry):
        c0 = r * _G
        gd = [
            pltpu.async_copy(table_hbm.at[idx_v.at[c0 + g]], rows_v.at[g], gsem)
            for g in range(_G)
        ]
        for d in gd:
            d.wait()
        sd = [
            pltpu.async_copy(
                rows_v.at[g], out_hbm.at[pl.ds(base + (c0 + g) * _CH, _CH)], ssem
            )
            for g in range(_G)
        ]
        for d in sd:
            d.wait()
        return carry

    lax.fori_loop(0, _NR, round_body, 0)


def kernel(token_ids, embedding_weight):
    idx = token_ids.astype(jnp.int32).reshape(_NW, _NCH, _CH)
    out = _sc_gather(idx, embedding_weight)
    return out.reshape(_BATCH, _HIST, _D)


# speedup vs baseline: 1.0065x; 1.0065x over previous
import functools

import jax
import jax.numpy as jnp
from jax import lax
from jax.experimental import pallas as pl
from jax.experimental.pallas import tpu as pltpu
from jax.experimental.pallas import tpu_sc as plsc

_VOCAB = 1000000
_D = 64
_BATCH = 4096
_HIST = 200

_NC, _NS = 2, 16
_NW = _NC * _NS
_B = _BATCH * _HIST
_BPW = _B // _NW
_CH = 128
_NCH = _BPW // _CH
_G = 4
_NG = _NCH // _G
_GR = _G * _CH

_mesh = plsc.VectorSubcoreMesh(core_axis_name="c", subcore_axis_name="s")


@functools.partial(
    pl.kernel,
    mesh=_mesh,
    out_type=jax.ShapeDtypeStruct((_B, _D), jnp.float32),
    scratch_types=[
        pltpu.VMEM((_NCH, _CH), jnp.int32),
        pltpu.VMEM((2, _GR, _D), jnp.float32),
        pltpu.SemaphoreType.DMA,
        pltpu.SemaphoreType.DMA,
        pltpu.SemaphoreType.DMA,
        pltpu.SemaphoreType.DMA,
    ],
    compiler_params=pltpu.CompilerParams(use_tc_tiling_on_sc=False),
)
def _sc_gather(idx_hbm, table_hbm, out_hbm, idx_v, rows_v, g0, g1, s0, s1):
    wid = lax.axis_index("s") * _NC + lax.axis_index("c")
    base = wid * _BPW
    pltpu.sync_copy(idx_hbm.at[wid], idx_v)

    gsem = (g0, g1)
    ssem = (s0, s1)

    def fire_ga(g, b):
        for j in range(_G):
            pltpu.async_copy(
                table_hbm.at[idx_v.at[g * _G + j]],
                rows_v.at[b, pl.ds(j * _CH, _CH)],
                gsem[b],
            )

    def drain_ga(g, b):
        for j in range(_G):
            pltpu.make_async_copy(
                table_hbm.at[idx_v.at[g * _G + j]],
                rows_v.at[b, pl.ds(j * _CH, _CH)],
                gsem[b],
            ).wait()

    def fire_st(g, b):
        pltpu.async_copy(
            rows_v.at[b], out_hbm.at[pl.ds(base + g * _GR, _GR)], ssem[b]
        )

    def drain_st(g, b):
        pltpu.make_async_copy(
            rows_v.at[b], out_hbm.at[pl.ds(base + g * _GR, _GR)], ssem[b]
        ).wait()

    fire_ga(0, 0)
    fire_ga(1, 1)
    drain_ga(0, 0)
    fire_st(0, 0)
    drain_st(0, 0)
    fire_ga(2, 0)
    drain_ga(1, 1)
    fire_st(1, 1)

    def body(h, carry):
        ge = 2 * h
        drain_st(ge - 1, 1)
        fire_ga(ge + 1, 1)
        drain_ga(ge, 0)
        fire_st(ge, 0)
        drain_st(ge, 0)
        fire_ga(ge + 2, 0)
        drain_ga(ge + 1, 1)
        fire_st(ge + 1, 1)
        return carry

    lax.fori_loop(1, _NG // 2 - 1, body, 0)

    ge = _NG - 2
    drain_st(ge - 1, 1)
    fire_ga(ge + 1, 1)
    drain_ga(ge, 0)
    fire_st(ge, 0)
    drain_st(ge, 0)
    drain_ga(ge + 1, 1)
    fire_st(ge + 1, 1)
    drain_st(ge + 1, 1)


def kernel(token_ids, embedding_weight):
    idx = token_ids.astype(jnp.int32).reshape(_NW, _NCH, _CH)
    out = _sc_gather(idx, embedding_weight)
    return out.reshape(_BATCH, _HIST, _D)
